# baseline (speedup 1.0000x reference)
import jax
import jax.numpy as jnp
from jax.experimental import pallas as pl
from jax.experimental.pallas import tpu as pltpu

K = 8
NEG = -1e30
INV_LN2 = 1.4426950408889634
ROW_BLOCK = 2048
STRIPS = 4


def _ladd2(x, y):
    m = jnp.maximum(x, y)
    d = jnp.minimum(x, y) - m
    return m + jnp.log2(1.0 + jnp.exp2(d))


def _simple_kernel(scores_ref, u_ref, mask_ref, marg_ref,
                   lwT_ref, f_ref, c_ref, pert_ref):
    R = scores_ref.shape[0]
    N = scores_ref.shape[1]
    S = STRIPS
    H = R // S

    lwT = scores_ref[...].T
    lwT_ref[...] = lwT * INV_LN2

    zero_row = jnp.zeros((1, H), jnp.float32)
    neg_row = jnp.full((1, H), NEG, jnp.float32)

    einit = jnp.full((K, H), NEG, jnp.float32)
    dinit = jnp.concatenate([jnp.full((K - 1, H), NEG, jnp.float32), zero_row],
                            axis=0)

    def body(t, carry):
        es, ds = carry
        s = N - 1 - t
        af = lwT_ref[pl.ds(t, 1), :]
        ab = lwT_ref[pl.ds(s, 1), :]
        new_es = []
        new_ds = []
        for c in range(S):
            sl = slice(c * H, (c + 1) * H)
            e, d = es[c], ds[c]
            fslab = jnp.concatenate([zero_row, e[:-1]], axis=0)
            f_ref[t, :, sl] = fslab
            c_ref[s, :, sl] = d
            new_es.append(_ladd2(e, fslab + af[:, sl]))
            new_ds.append(
                _ladd2(d, jnp.concatenate([d[1:], neg_row], 0) + ab[:, sl]))
        return tuple(new_es), tuple(new_ds)

    es, _ = jax.lax.fori_loop(
        0, N, body, ((einit,) * S, (dinit,) * S), unroll=2)
    log_z = jnp.concatenate([es[c][K - 1:K, :] for c in range(S)],
                            axis=1)

    z = f_ref[...] + c_ref[...]
    m = jnp.max(z, axis=1)
    le = m + jnp.log2(jnp.sum(jnp.exp2(z - m[:, None, :]), axis=1))

    marg_t = jnp.exp2(lwT_ref[...] + le - log_z)
    marg_ref[...] = marg_t.T

    pert_ref[...] = lwT + (-jnp.log(-jnp.log(u_ref[...].T)))
    iota = jax.lax.broadcasted_iota(jnp.int32, (N, R), 0)
    hard_t = jnp.zeros((N, R), jnp.float32)
    for _ in range(K):
        pert = pert_ref[...]
        mx = jnp.max(pert, axis=0, keepdims=True)
        eq = pert == mx
        idx = jnp.min(jnp.where(eq, iota, N), axis=0, keepdims=True)
        sel = iota == idx
        hard_t = hard_t + sel.astype(jnp.float32)
        pert_ref[...] = jnp.where(sel, -jnp.inf, pert)

    mask_ref[...] = ((hard_t - marg_t) + marg_t).T


def kernel(scores):
    bsz, window, ensemble = scores.shape
    rows = bsz * window
    flat = scores.reshape(rows, ensemble)

    gkey = jax.random.key(42)
    u = jax.random.uniform(gkey, (1, rows, ensemble), minval=1e-9, maxval=1.0,
                           dtype=jnp.float32)
    u = u.reshape(rows, ensemble)

    R = ROW_BLOCK
    grid = (rows // R,)
    mask, marg = pl.pallas_call(
        _simple_kernel,
        grid=grid,
        in_specs=[
            pl.BlockSpec((R, ensemble), lambda i: (i, 0)),
            pl.BlockSpec((R, ensemble), lambda i: (i, 0)),
        ],
        out_specs=[
            pl.BlockSpec((R, ensemble), lambda i: (i, 0)),
            pl.BlockSpec((R, ensemble), lambda i: (i, 0)),
        ],
        out_shape=[
            jax.ShapeDtypeStruct((rows, ensemble), jnp.float32),
            jax.ShapeDtypeStruct((rows, ensemble), jnp.float32),
        ],
        scratch_shapes=[
            pltpu.VMEM((ensemble, R), jnp.float32),
            pltpu.VMEM((ensemble, K, R), jnp.float32),
            pltpu.VMEM((ensemble, K, R), jnp.float32),
            pltpu.VMEM((ensemble, R), jnp.float32),
        ],
    )(flat, u)

    new_mask = mask.reshape(bsz, window, ensemble)
    new_marginals = marg.reshape(bsz, window, ensemble)
    return new_mask, new_marginals


# speedup vs baseline: 1.2310x; 1.2310x over previous
import jax
import jax.numpy as jnp
from jax.experimental import pallas as pl
from jax.experimental.pallas import tpu as pltpu

K = 8
NEG = -1e30
INV_LN2 = 1.4426950408889634
ROW_BLOCK = 2048
STRIPS = 4


def _pert_chunk(base, h, lw2_chunk):
    n = lw2_chunk.shape[0]
    sub = jax.lax.broadcasted_iota(jnp.uint32, (n, h), 0)
    lane = jax.lax.broadcasted_iota(jnp.uint32, (n, h), 1)
    x1 = base + lane * jnp.uint32(n) + sub
    ks0 = jnp.uint32(0)
    ks1 = jnp.uint32(42)
    ks2 = jnp.uint32(0x1BD11BDA ^ 42)
    rots = ((13, 15, 26, 6), (17, 29, 16, 24))
    keys = (ks0, ks1, ks2)
    x0 = jnp.full((n, h), ks0, jnp.uint32)
    x1 = x1 + ks1
    for i in range(5):
        for r in rots[i % 2]:
            x0 = x0 + x1
            x1 = jax.lax.shift_left(x1, jnp.uint32(r)) | \
                jax.lax.shift_right_logical(x1, jnp.uint32(32 - r))
            x1 = x1 ^ x0
        x0 = x0 + keys[(i + 1) % 3]
        x1 = x1 + keys[(i + 2) % 3] + jnp.uint32(i + 1)
    bits = x0 ^ x1
    fb = jax.lax.shift_right_logical(bits, jnp.uint32(9)) | \
        jnp.uint32(0x3F800000)
    f = jax.lax.bitcast_convert_type(fb, jnp.float32) - jnp.float32(1.0)
    mn = jnp.float32(1e-9)
    u = jnp.maximum(mn, f * (jnp.float32(1.0) - mn) + mn)
    return lw2_chunk + (-jnp.log(-jnp.log(u)))


def _ladd2(x, y):
    m = jnp.maximum(x, y)
    d = jnp.minimum(x, y) - m
    return m + jnp.log2(1.0 + jnp.exp2(d))


def _simple_kernel(scores_ref, mask_ref, marg_ref,
                   lwT_ref, f_ref, c_ref, pert_ref):
    R = scores_ref.shape[0]
    N = scores_ref.shape[1]
    S = STRIPS
    H = R // S

    lwT = scores_ref[...].T
    lwT_ref[...] = lwT * INV_LN2

    zero_row = jnp.zeros((1, H), jnp.float32)
    neg_row = jnp.full((1, H), NEG, jnp.float32)

    einit = jnp.full((K, H), NEG, jnp.float32)
    dinit = jnp.concatenate([jnp.full((K - 1, H), NEG, jnp.float32), zero_row],
                            axis=0)

    def body(t, carry):
        es, ds = carry
        s = N - 1 - t
        af = lwT_ref[pl.ds(t, 1), :]
        ab = lwT_ref[pl.ds(s, 1), :]
        new_es = []
        new_ds = []
        for c in range(S):
            sl = slice(c * H, (c + 1) * H)
            e, d = es[c], ds[c]
            fslab = jnp.concatenate([zero_row, e[:-1]], axis=0)
            f_ref[t, :, sl] = fslab
            c_ref[s, :, sl] = d
            new_es.append(_ladd2(e, fslab + af[:, sl]))
            new_ds.append(
                _ladd2(d, jnp.concatenate([d[1:], neg_row], 0) + ab[:, sl]))
        return tuple(new_es), tuple(new_ds)

    es, _ = jax.lax.fori_loop(
        0, N, body, ((einit,) * S, (dinit,) * S), unroll=2)
    log_z = jnp.concatenate([es[c][K - 1:K, :] for c in range(S)],
                            axis=1)

    sh = lwT_ref[...] - log_z
    z = f_ref[...] + c_ref[...]
    marg_t = jnp.sum(jnp.exp2(z + sh[:, None, :]), axis=1)
    marg_ref[...] = marg_t.T

    CH = 256
    pid = pl.program_id(0)
    for c in range(R // CH):
        base = (pid.astype(jnp.uint32) * jnp.uint32(R)
                + jnp.uint32(c * CH)) * jnp.uint32(N)
        pert_ref[:, c * CH:(c + 1) * CH] = _pert_chunk(
            base, CH, lwT[:, c * CH:(c + 1) * CH])
    iota = jax.lax.broadcasted_iota(jnp.int32, (N, R), 0)
    hard_t = jnp.zeros((N, R), jnp.float32)
    for _ in range(K):
        pert = pert_ref[...]
        mx = jnp.max(pert, axis=0, keepdims=True)
        eq = pert == mx
        idx = jnp.min(jnp.where(eq, iota, N), axis=0, keepdims=True)
        sel = iota == idx
        hard_t = hard_t + sel.astype(jnp.float32)
        pert_ref[...] = jnp.where(sel, -jnp.inf, pert)

    mask_ref[...] = ((hard_t - marg_t) + marg_t).T


def kernel(scores):
    bsz, window, ensemble = scores.shape
    rows = bsz * window
    flat = scores.reshape(rows, ensemble)

    R = ROW_BLOCK
    grid = (rows // R,)
    mask, marg = pl.pallas_call(
        _simple_kernel,
        grid=grid,
        in_specs=[
            pl.BlockSpec((R, ensemble), lambda i: (i, 0)),
        ],
        out_specs=[
            pl.BlockSpec((R, ensemble), lambda i: (i, 0)),
            pl.BlockSpec((R, ensemble), lambda i: (i, 0)),
        ],
        out_shape=[
            jax.ShapeDtypeStruct((rows, ensemble), jnp.float32),
            jax.ShapeDtypeStruct((rows, ensemble), jnp.float32),
        ],
        scratch_shapes=[
            pltpu.VMEM((ensemble, R), jnp.float32),
            pltpu.VMEM((ensemble, K, R), jnp.float32),
            pltpu.VMEM((ensemble, K, R), jnp.float32),
            pltpu.VMEM((ensemble, R), jnp.float32),
        ],
    )(flat)

    new_mask = mask.reshape(bsz, window, ensemble)
    new_marginals = marg.reshape(bsz, window, ensemble)
    return new_mask, new_marginals


# speedup vs baseline: 1.2568x; 1.0210x over previous
import jax
import jax.numpy as jnp
from jax.experimental import pallas as pl
from jax.experimental.pallas import tpu as pltpu

K = 8
NEG = -1e30
INV_LN2 = 1.4426950408889634
ROW_BLOCK = 2048
STRIPS = 4


def _pert_chunk(base, h, lw2_chunk):
    n = lw2_chunk.shape[0]
    sub = jax.lax.broadcasted_iota(jnp.uint32, (n, h), 0)
    lane = jax.lax.broadcasted_iota(jnp.uint32, (n, h), 1)
    x1 = base + lane * jnp.uint32(n) + sub
    ks0 = jnp.uint32(0)
    ks1 = jnp.uint32(42)
    ks2 = jnp.uint32(0x1BD11BDA ^ 42)
    rots = ((13, 15, 26, 6), (17, 29, 16, 24))
    keys = (ks0, ks1, ks2)
    x0 = jnp.full((n, h), ks0, jnp.uint32)
    x1 = x1 + ks1
    for i in range(5):
        for r in rots[i % 2]:
            x0 = x0 + x1
            x1 = jax.lax.shift_left(x1, jnp.uint32(r)) | \
                jax.lax.shift_right_logical(x1, jnp.uint32(32 - r))
            x1 = x1 ^ x0
        x0 = x0 + keys[(i + 1) % 3]
        x1 = x1 + keys[(i + 2) % 3] + jnp.uint32(i + 1)
    bits = x0 ^ x1
    fb = jax.lax.shift_right_logical(bits, jnp.uint32(9)) | \
        jnp.uint32(0x3F800000)
    f = jax.lax.bitcast_convert_type(fb, jnp.float32) - jnp.float32(1.0)
    mn = jnp.float32(1e-9)
    u = jnp.maximum(mn, f * (jnp.float32(1.0) - mn) + mn)
    return lw2_chunk + (-jnp.log(-jnp.log(u)))


def _ladd2(x, y):
    m = jnp.maximum(x, y)
    d = jnp.minimum(x, y) - m
    return m + jnp.log2(1.0 + jnp.exp2(d))


def _simple_kernel(scores_ref, mask_ref, marg_ref,
                   lwT_ref, f_ref, c_ref, pert_ref):
    R = scores_ref.shape[0]
    N = scores_ref.shape[1]
    S = STRIPS
    H = R // S

    lwT = scores_ref[...].T
    lwT_ref[...] = lwT * INV_LN2

    zero_row = jnp.zeros((1, H), jnp.float32)
    neg_row = jnp.full((1, H), NEG, jnp.float32)

    einit = jnp.full((K, H), NEG, jnp.float32)
    dinit = jnp.concatenate([jnp.full((K - 1, H), NEG, jnp.float32), zero_row],
                            axis=0)

    def body(t, carry):
        es, ds = carry
        s = N - 1 - t
        af = lwT_ref[pl.ds(t, 1), :]
        ab = lwT_ref[pl.ds(s, 1), :]
        new_es = []
        new_ds = []
        for c in range(S):
            sl = slice(c * H, (c + 1) * H)
            e, d = es[c], ds[c]
            fslab = jnp.concatenate([zero_row, e[:-1]], axis=0)
            f_ref[t, :, sl] = fslab
            c_ref[s, :, sl] = d
            new_es.append(_ladd2(e, fslab + af[:, sl]))
            new_ds.append(
                _ladd2(d, jnp.concatenate([d[1:], neg_row], 0) + ab[:, sl]))
        return tuple(new_es), tuple(new_ds)

    es, _ = jax.lax.fori_loop(
        0, N, body, ((einit,) * S, (dinit,) * S), unroll=4)
    log_z = jnp.concatenate([es[c][K - 1:K, :] for c in range(S)],
                            axis=1)

    sh = lwT_ref[...] - log_z
    z = f_ref[...] + c_ref[...]
    marg_t = jnp.sum(jnp.exp2(z + sh[:, None, :]), axis=1)
    marg_ref[...] = marg_t.T

    CH = 256
    pid = pl.program_id(0)
    for c in range(R // CH):
        base = (pid.astype(jnp.uint32) * jnp.uint32(R)
                + jnp.uint32(c * CH)) * jnp.uint32(N)
        pert_ref[:, c * CH:(c + 1) * CH] = _pert_chunk(
            base, CH, lwT[:, c * CH:(c + 1) * CH])
    iota = jax.lax.broadcasted_iota(jnp.int32, (N, R), 0)
    hard_t = jnp.zeros((N, R), jnp.float32)
    for _ in range(K):
        pert = pert_ref[...]
        mx = jnp.max(pert, axis=0, keepdims=True)
        eq = pert == mx
        idx = jnp.min(jnp.where(eq, iota, N), axis=0, keepdims=True)
        sel = iota == idx
        hard_t = hard_t + sel.astype(jnp.float32)
        pert_ref[...] = jnp.where(sel, -jnp.inf, pert)

    mask_ref[...] = ((hard_t - marg_t) + marg_t).T


def kernel(scores):
    bsz, window, ensemble = scores.shape
    rows = bsz * window
    flat = scores.reshape(rows, ensemble)

    R = ROW_BLOCK
    grid = (rows // R,)
    mask, marg = pl.pallas_call(
        _simple_kernel,
        grid=grid,
        in_specs=[
            pl.BlockSpec((R, ensemble), lambda i: (i, 0)),
        ],
        out_specs=[
            pl.BlockSpec((R, ensemble), lambda i: (i, 0)),
            pl.BlockSpec((R, ensemble), lambda i: (i, 0)),
        ],
        out_shape=[
            jax.ShapeDtypeStruct((rows, ensemble), jnp.float32),
            jax.ShapeDtypeStruct((rows, ensemble), jnp.float32),
        ],
        scratch_shapes=[
            pltpu.VMEM((ensemble, R), jnp.float32),
            pltpu.VMEM((ensemble, K, R), jnp.float32),
            pltpu.VMEM((ensemble, K, R), jnp.float32),
            pltpu.VMEM((ensemble, R), jnp.float32),
        ],
    )(flat)

    new_mask = mask.reshape(bsz, window, ensemble)
    new_marginals = marg.reshape(bsz, window, ensemble)
    return new_mask, new_marginals


# speedup vs baseline: 1.2624x; 1.0044x over previous
import jax
import jax.numpy as jnp
from jax.experimental import pallas as pl
from jax.experimental.pallas import tpu as pltpu

K = 8
NEG = -1e30
INV_LN2 = 1.4426950408889634
ROW_BLOCK = 2048
STRIPS = 4


def _pert_chunk(base, h, lw2_chunk):
    n = lw2_chunk.shape[0]
    sub = jax.lax.broadcasted_iota(jnp.uint32, (n, h), 0)
    lane = jax.lax.broadcasted_iota(jnp.uint32, (n, h), 1)
    x1 = base + lane * jnp.uint32(n) + sub
    ks0 = jnp.uint32(0)
    ks1 = jnp.uint32(42)
    ks2 = jnp.uint32(0x1BD11BDA ^ 42)
    rots = ((13, 15, 26, 6), (17, 29, 16, 24))
    keys = (ks0, ks1, ks2)
    x0 = jnp.full((n, h), ks0, jnp.uint32)
    x1 = x1 + ks1
    for i in range(5):
        for r in rots[i % 2]:
            x0 = x0 + x1
            x1 = jax.lax.shift_left(x1, jnp.uint32(r)) | \
                jax.lax.shift_right_logical(x1, jnp.uint32(32 - r))
            x1 = x1 ^ x0
        x0 = x0 + keys[(i + 1) % 3]
        x1 = x1 + keys[(i + 2) % 3] + jnp.uint32(i + 1)
    bits = x0 ^ x1
    fb = jax.lax.shift_right_logical(bits, jnp.uint32(9)) | \
        jnp.uint32(0x3F800000)
    f = jax.lax.bitcast_convert_type(fb, jnp.float32) - jnp.float32(1.0)
    mn = jnp.float32(1e-9)
    u = jnp.maximum(mn, f * (jnp.float32(1.0) - mn) + mn)
    return lw2_chunk + (-jnp.log(-jnp.log(u)))


def _ladd2(x, y):
    m = jnp.maximum(x, y)
    d = jnp.minimum(x, y) - m
    return m + jnp.log2(1.0 + jnp.exp2(d))


def _simple_kernel(scores_ref, mask_ref, marg_ref,
                   lwT_ref, f_ref, c_ref, pert_ref):
    R = scores_ref.shape[0]
    N = scores_ref.shape[1]
    S = STRIPS
    H = R // S

    lwT = scores_ref[...].T
    lwT_ref[...] = lwT * INV_LN2

    zero_row = jnp.zeros((1, H), jnp.float32)
    neg_row = jnp.full((1, H), NEG, jnp.float32)

    einit = jnp.full((K, H), NEG, jnp.float32)
    dinit = jnp.concatenate([jnp.full((K - 1, H), NEG, jnp.float32), zero_row],
                            axis=0)

    def body(t, carry):
        es, ds = carry
        s = N - 1 - t
        af = lwT_ref[pl.ds(t, 1), :]
        ab = lwT_ref[pl.ds(s, 1), :]
        new_es = []
        new_ds = []
        for c in range(S):
            sl = slice(c * H, (c + 1) * H)
            e, d = es[c], ds[c]
            fslab = jnp.concatenate([zero_row, e[:-1]], axis=0)
            f_ref[t, :, sl] = fslab
            c_ref[s, :, sl] = d
            new_es.append(_ladd2(e, fslab + af[:, sl]))
            new_ds.append(
                _ladd2(d, jnp.concatenate([d[1:], neg_row], 0) + ab[:, sl]))
        return tuple(new_es), tuple(new_ds)

    es, _ = jax.lax.fori_loop(
        0, N, body, ((einit,) * S, (dinit,) * S), unroll=8)
    log_z = jnp.concatenate([es[c][K - 1:K, :] for c in range(S)],
                            axis=1)

    sh = lwT_ref[...] - log_z
    z = f_ref[...] + c_ref[...]
    marg_t = jnp.sum(jnp.exp2(z + sh[:, None, :]), axis=1)
    marg_ref[...] = marg_t.T

    CH = 256
    pid = pl.program_id(0)
    for c in range(R // CH):
        base = (pid.astype(jnp.uint32) * jnp.uint32(R)
                + jnp.uint32(c * CH)) * jnp.uint32(N)
        pert_ref[:, c * CH:(c + 1) * CH] = _pert_chunk(
            base, CH, lwT[:, c * CH:(c + 1) * CH])
    iota = jax.lax.broadcasted_iota(jnp.int32, (N, R), 0)
    hard_t = jnp.zeros((N, R), jnp.float32)
    for _ in range(K):
        pert = pert_ref[...]
        mx = jnp.max(pert, axis=0, keepdims=True)
        eq = pert == mx
        idx = jnp.min(jnp.where(eq, iota, N), axis=0, keepdims=True)
        sel = iota == idx
        hard_t = hard_t + sel.astype(jnp.float32)
        pert_ref[...] = jnp.where(sel, -jnp.inf, pert)

    mask_ref[...] = ((hard_t - marg_t) + marg_t).T


def kernel(scores):
    bsz, window, ensemble = scores.shape
    rows = bsz * window
    flat = scores.reshape(rows, ensemble)

    R = ROW_BLOCK
    grid = (rows // R,)
    mask, marg = pl.pallas_call(
        _simple_kernel,
        grid=grid,
        in_specs=[
            pl.BlockSpec((R, ensemble), lambda i: (i, 0)),
        ],
        out_specs=[
            pl.BlockSpec((R, ensemble), lambda i: (i, 0)),
            pl.BlockSpec((R, ensemble), lambda i: (i, 0)),
        ],
        out_shape=[
            jax.ShapeDtypeStruct((rows, ensemble), jnp.float32),
            jax.ShapeDtypeStruct((rows, ensemble), jnp.float32),
        ],
        scratch_shapes=[
            pltpu.VMEM((ensemble, R), jnp.float32),
            pltpu.VMEM((ensemble, K, R), jnp.float32),
            pltpu.VMEM((ensemble, K, R), jnp.float32),
            pltpu.VMEM((ensemble, R), jnp.float32),
        ],
    )(flat)

    new_mask = mask.reshape(bsz, window, ensemble)
    new_marginals = marg.reshape(bsz, window, ensemble)
    return new_mask, new_marginals
